# Initial kernel scaffold; baseline (speedup 1.0000x reference)
import functools

import jax
import jax.numpy as jnp
from jax import lax
from jax.experimental import pallas as pl
from jax.experimental.pallas import tpu as pltpu
from jax.experimental.pallas import tpu_sc as plsc

H, W, C = 64, 64, 256
PH, PW = 7, 7
N = 1000

NC, NS = 2, 16
NW = NC * NS
BLK = 16
NB = (N + BLK - 1) // BLK
BLOCKS_PER_W = (NB + NW - 1) // NW
ROWS_PER_BLK = BLK * PH * PW
CHUNK = 56
NCH = ROWS_PER_BLK // CHUNK
TOTAL_ROWS = N * PH * PW
NPAD = NB * BLK


def _body(fm, props, out, props_v, idx_v, gbuf, gsem):
    wid = lax.axis_index("s") * NC + lax.axis_index("c")
    pltpu.sync_copy(props, props_v)
    pos_base = lax.iota(jnp.int32, 16) * (PH * PW)

    for t in range(BLOCKS_PER_W):
        b = wid + NW * t

        @pl.when(b < NB)
        def _process_block():
            pb = b * BLK
            x0 = props_v[0, pl.ds(pb, BLK)]
            y0 = props_v[1, pl.ds(pb, BLK)]
            x1 = props_v[2, pl.ds(pb, BLK)]
            y1 = props_v[3, pl.ds(pb, BLK)]
            xmin = x0.astype(jnp.int32)
            ymin = y0.astype(jnp.int32)
            xt = x1.astype(jnp.int32)
            yt = y1.astype(jnp.int32)
            xmax = jnp.where(x1 > xt.astype(jnp.float32), xt + 1, xt)
            ymax = jnp.where(y1 > yt.astype(jnp.float32), yt + 1, yt)
            hh = jnp.maximum(ymax - ymin, 1)
            ww = jnp.maximum(xmax - xmin, 1)
            rs, cs = [], []
            for i in range(PH):
                r = jnp.minimum(((2 * i + 1) * hh) // (2 * PH), hh - 1) + ymin
                rs.append(jnp.clip(r, 0, H - 1))
            for j in range(PW):
                c = jnp.minimum(((2 * j + 1) * ww) // (2 * PW), ww - 1) + xmin
                cs.append(jnp.clip(c, 0, W - 1))
            for i in range(PH):
                rW = rs[i] * W
                for j in range(PW):
                    plsc.store_scatter(idx_v, [pos_base + (i * PW + j)], rW + cs[j])

            row0 = b * ROWS_PER_BLK
            copies = [None] * NCH
            for cc in range(NCH):
                @pl.when(row0 + cc * CHUNK < TOTAL_ROWS)
                def _start(cc=cc):
                    copies[cc] = pltpu.async_copy(
                        fm.at[idx_v.at[pl.ds(cc * CHUNK, CHUNK)]],
                        gbuf.at[cc % 2], gsem)
                if cc > 0:
                    @pl.when(row0 + (cc - 1) * CHUNK < TOTAL_ROWS)
                    def _drain(cc=cc):
                        copies[cc - 1].wait()
                        pltpu.sync_copy(
                            gbuf.at[(cc - 1) % 2],
                            out.at[pl.ds(row0 + (cc - 1) * CHUNK, CHUNK)])

            @pl.when(row0 + (NCH - 1) * CHUNK < TOTAL_ROWS)
            def _last():
                copies[NCH - 1].wait()
                pltpu.sync_copy(
                    gbuf.at[(NCH - 1) % 2],
                    out.at[pl.ds(row0 + (NCH - 1) * CHUNK, CHUNK)])


_sc_gather = functools.partial(
    pl.kernel,
    out_type=jax.ShapeDtypeStruct((TOTAL_ROWS, C), jnp.float32),
    mesh=plsc.VectorSubcoreMesh(
        core_axis_name="c", subcore_axis_name="s",
        num_cores=NC, num_subcores=NS),
    scratch_types=[
        pltpu.VMEM((4, NPAD), jnp.float32),
        pltpu.VMEM((ROWS_PER_BLK,), jnp.int32),
        pltpu.VMEM((2, CHUNK, C), jnp.float32),
        pltpu.SemaphoreType.DMA,
    ],
)(_body)


@jax.jit
def kernel(feature_map, proposals):
    fm = feature_map.reshape(H * W, C)
    p = proposals[0]
    props = jnp.zeros((4, NPAD), jnp.float32).at[:, :N].set(p.T)
    out = _sc_gather(fm, props)
    return out.reshape(1, N, PH, PW, C)


# speedup vs baseline: 2.6606x; 2.6606x over previous
import functools

import jax
import jax.numpy as jnp
from jax import lax
from jax.experimental import pallas as pl
from jax.experimental.pallas import tpu as pltpu
from jax.experimental.pallas import tpu_sc as plsc

H, W, C = 64, 64, 256
PH, PW = 7, 7
N = 1000

NC, NS = 2, 16
NW = NC * NS
BLK = 8
NB = N // BLK
BLOCKS_PER_W = -(-NB // NW)
ROWS_PER_BLK = BLK * PH * PW
CHUNK = 56
NCH = ROWS_PER_BLK // CHUNK
TOTAL_ROWS = N * PH * PW
NPAD = NB * BLK + 16


def _body(fm, props, out, props_v, idx_v, gbuf, gsem):
    wid = lax.axis_index("s") * NC + lax.axis_index("c")
    pltpu.sync_copy(props, props_v)

    for t in range(BLOCKS_PER_W):
        b = wid + NW * t

        @pl.when(b < NB)
        def _process_block():
            pos_base = lax.iota(jnp.int32, 16) * (PH * PW)
            pb = b * BLK
            x0 = props_v[0, pl.ds(pb, 16)]
            y0 = props_v[1, pl.ds(pb, 16)]
            x1 = props_v[2, pl.ds(pb, 16)]
            y1 = props_v[3, pl.ds(pb, 16)]
            xmin = x0.astype(jnp.int32)
            ymin = y0.astype(jnp.int32)
            xt = x1.astype(jnp.int32)
            yt = y1.astype(jnp.int32)
            xmax = jnp.where(x1 > xt.astype(jnp.float32), xt + 1, xt)
            ymax = jnp.where(y1 > yt.astype(jnp.float32), yt + 1, yt)
            hh = jnp.maximum(ymax - ymin, 1)
            ww = jnp.maximum(xmax - xmin, 1)
            rs, cs = [], []
            for i in range(PH):
                r = jnp.minimum(((2 * i + 1) * hh) // (2 * PH), hh - 1) + ymin
                rs.append(jnp.clip(r, 0, H - 1))
            for j in range(PW):
                c = jnp.minimum(((2 * j + 1) * ww) // (2 * PW), ww - 1) + xmin
                cs.append(jnp.clip(c, 0, W - 1))
            for i in range(PH):
                rW = rs[i] * W
                for j in range(PW):
                    plsc.store_scatter(idx_v, [pos_base + (i * PW + j)], rW + cs[j])

            row0 = b * ROWS_PER_BLK
            copies = [None] * NCH
            for cc in range(NCH):
                copies[cc] = pltpu.async_copy(
                    fm.at[idx_v.at[pl.ds(cc * CHUNK, CHUNK)]],
                    gbuf.at[cc % 2], gsem.at[cc % 2])
                if cc > 0:
                    copies[cc - 1].wait()
                    pltpu.sync_copy(
                        gbuf.at[(cc - 1) % 2],
                        out.at[pl.ds(row0 + (cc - 1) * CHUNK, CHUNK)])
            copies[NCH - 1].wait()
            pltpu.sync_copy(
                gbuf.at[(NCH - 1) % 2],
                out.at[pl.ds(row0 + (NCH - 1) * CHUNK, CHUNK)])


_sc_gather = functools.partial(
    pl.kernel,
    out_type=jax.ShapeDtypeStruct((TOTAL_ROWS, C), jnp.float32),
    mesh=plsc.VectorSubcoreMesh(
        core_axis_name="c", subcore_axis_name="s",
        num_cores=NC, num_subcores=NS),
    scratch_types=[
        pltpu.VMEM((4, NPAD), jnp.float32),
        pltpu.VMEM((2 * ROWS_PER_BLK,), jnp.int32),
        pltpu.VMEM((2, CHUNK, C), jnp.float32),
        pltpu.SemaphoreType.DMA((2,)),
    ],
    compiler_params=pltpu.CompilerParams(needs_layout_passes=False),
)(_body)


@jax.jit
def kernel(feature_map, proposals):
    fm = feature_map.reshape(H * W, C)
    p = proposals[0]
    props = jnp.zeros((4, NPAD), jnp.float32).at[:, :N].set(p.T)
    out = _sc_gather(fm, props)
    return out.reshape(1, N, PH, PW, C)


# speedup vs baseline: 2.6858x; 1.0095x over previous
import functools

import jax
import jax.numpy as jnp
from jax import lax
from jax.experimental import pallas as pl
from jax.experimental.pallas import tpu as pltpu
from jax.experimental.pallas import tpu_sc as plsc

H, W, C = 64, 64, 256
PH, PW = 7, 7
N = 1000

NC, NS = 2, 16
NW = NC * NS
BLK = 8
NB = N // BLK
BLOCKS_PER_W = -(-NB // NW)
ROWS_PER_BLK = BLK * PH * PW
CHUNKS = ((0, 112), (112, 112), (224, 112), (336, 56))
TOTAL_ROWS = N * PH * PW
NPAD = NB * BLK + 16


def _body(fm, props, out, props_v, idx_v, gbuf, gsem):
    wid = lax.axis_index("s") * NC + lax.axis_index("c")
    pltpu.sync_copy(props, props_v)

    for t in range(BLOCKS_PER_W):
        b = wid + NW * t

        @pl.when(b < NB)
        def _process_block():
            pos_base = lax.iota(jnp.int32, 16) * (PH * PW)
            pb = b * BLK
            x0 = props_v[0, pl.ds(pb, 16)]
            y0 = props_v[1, pl.ds(pb, 16)]
            x1 = props_v[2, pl.ds(pb, 16)]
            y1 = props_v[3, pl.ds(pb, 16)]
            xmin = x0.astype(jnp.int32)
            ymin = y0.astype(jnp.int32)
            xt = x1.astype(jnp.int32)
            yt = y1.astype(jnp.int32)
            xmax = jnp.where(x1 > xt.astype(jnp.float32), xt + 1, xt)
            ymax = jnp.where(y1 > yt.astype(jnp.float32), yt + 1, yt)
            hh = jnp.maximum(ymax - ymin, 1)
            ww = jnp.maximum(xmax - xmin, 1)
            rs, cs = [], []
            for i in range(PH):
                r = jnp.minimum(((2 * i + 1) * hh) // (2 * PH), hh - 1) + ymin
                rs.append(jnp.clip(r, 0, H - 1))
            for j in range(PW):
                c = jnp.minimum(((2 * j + 1) * ww) // (2 * PW), ww - 1) + xmin
                cs.append(jnp.clip(c, 0, W - 1))
            for i in range(PH):
                rW = rs[i] * W
                for j in range(PW):
                    plsc.store_scatter(idx_v, [pos_base + (i * PW + j)], rW + cs[j])

            row0 = b * ROWS_PER_BLK
            copies = [
                pltpu.async_copy(
                    fm.at[idx_v.at[pl.ds(off, n)]],
                    gbuf.at[pl.ds(off, n)], gsem)
                for off, n in CHUNKS
            ]
            for cp in copies:
                cp.wait()
            pltpu.sync_copy(gbuf, out.at[pl.ds(row0, ROWS_PER_BLK)])


_sc_gather = functools.partial(
    pl.kernel,
    out_type=jax.ShapeDtypeStruct((TOTAL_ROWS, C), jnp.float32),
    mesh=plsc.VectorSubcoreMesh(
        core_axis_name="c", subcore_axis_name="s",
        num_cores=NC, num_subcores=NS),
    scratch_types=[
        pltpu.VMEM((4, NPAD), jnp.float32),
        pltpu.VMEM((2 * ROWS_PER_BLK,), jnp.int32),
        pltpu.VMEM((ROWS_PER_BLK, C), jnp.float32),
        pltpu.SemaphoreType.DMA,
    ],
    compiler_params=pltpu.CompilerParams(needs_layout_passes=False),
)(_body)


@jax.jit
def kernel(feature_map, proposals):
    fm = feature_map.reshape(H * W, C)
    p = proposals[0]
    props = jnp.zeros((4, NPAD), jnp.float32).at[:, :N].set(p.T)
    out = _sc_gather(fm, props)
    return out.reshape(1, N, PH, PW, C)


# speedup vs baseline: 12.4125x; 4.6215x over previous
import functools

import jax
import jax.numpy as jnp
from jax import lax
from jax.experimental import pallas as pl
from jax.experimental.pallas import tpu as pltpu
from jax.experimental.pallas import tpu_sc as plsc

H, W, C = 64, 64, 256
PH, PW = 7, 7
N = 1000

NC, NS = 2, 16
NW = NC * NS
NSEG = PH * PW
CHUNK = 128
CPS = -(-N // CHUNK)
UNITS = NSEG * CPS
UNITS_PER_W = UNITS // NW
TOTAL_ROWS = NSEG * N
LAST_START = N - CHUNK


def _unit_indices(props_v, idx_v, u, par):
    s = u >> 3
    q = u & 7
    i = (s * 9363) >> 16
    j = s - i * PW
    p0 = jnp.minimum(q * CHUNK, LAST_START)
    for g in range(CHUNK // 16):
        off = p0 + g * 16
        x0 = props_v[0, pl.ds(off, 16)]
        y0 = props_v[1, pl.ds(off, 16)]
        x1 = props_v[2, pl.ds(off, 16)]
        y1 = props_v[3, pl.ds(off, 16)]
        xmin = x0.astype(jnp.int32)
        ymin = y0.astype(jnp.int32)
        xt = x1.astype(jnp.int32)
        yt = y1.astype(jnp.int32)
        xmax = jnp.where(x1 > xt.astype(jnp.float32), xt + 1, xt)
        ymax = jnp.where(y1 > yt.astype(jnp.float32), yt + 1, yt)
        hh = jnp.maximum(ymax - ymin, 1)
        ww = jnp.maximum(xmax - xmin, 1)
        r = jnp.minimum(((2 * i + 1) * hh * 4682) >> 16, hh - 1) + ymin
        r = jnp.clip(r, 0, H - 1)
        c = jnp.minimum(((2 * j + 1) * ww * 4682) >> 16, ww - 1) + xmin
        c = jnp.clip(c, 0, W - 1)
        idx_v[par, pl.ds(g * 16, 16)] = r * W + c
    return s * N + p0


def _body(fm, props, out, props_v, idx_v, gbuf, gsem):
    wid = lax.axis_index("s") * NC + lax.axis_index("c")
    pltpu.sync_copy(props, props_v)

    copies = [None] * UNITS_PER_W
    rows = [None] * UNITS_PER_W
    for t in range(UNITS_PER_W):
        par = t % 2
        rows[t] = _unit_indices(props_v, idx_v, wid + NW * t, par)
        copies[t] = pltpu.async_copy(
            fm.at[idx_v.at[par]], gbuf.at[par], gsem.at[par])
        if t > 0:
            copies[t - 1].wait()
            pltpu.sync_copy(gbuf.at[1 - par],
                            out.at[pl.ds(rows[t - 1], CHUNK)])
    copies[UNITS_PER_W - 1].wait()
    pltpu.sync_copy(gbuf.at[(UNITS_PER_W - 1) % 2],
                    out.at[pl.ds(rows[UNITS_PER_W - 1], CHUNK)])

    u = NW * UNITS_PER_W + wid

    @pl.when(u < UNITS)
    def _tail():
        row0 = _unit_indices(props_v, idx_v, u, 0)
        pltpu.async_copy(fm.at[idx_v.at[0]], gbuf.at[0], gsem.at[0]).wait()
        pltpu.sync_copy(gbuf.at[0], out.at[pl.ds(row0, CHUNK)])


_sc_gather = functools.partial(
    pl.kernel,
    out_type=jax.ShapeDtypeStruct((TOTAL_ROWS, C), jnp.float32),
    mesh=plsc.VectorSubcoreMesh(
        core_axis_name="c", subcore_axis_name="s",
        num_cores=NC, num_subcores=NS),
    scratch_types=[
        pltpu.VMEM((4, N), jnp.float32),
        pltpu.VMEM((2, CHUNK), jnp.int32),
        pltpu.VMEM((2, CHUNK, C), jnp.float32),
        pltpu.SemaphoreType.DMA((2,)),
    ],
    compiler_params=pltpu.CompilerParams(needs_layout_passes=False),
)(_body)


@jax.jit
def kernel(feature_map, proposals):
    fm = feature_map.reshape(H * W, C)
    props = proposals[0].T
    out = _sc_gather(fm, props)
    return jnp.transpose(
        out.reshape(PH, PW, N, C), (2, 0, 1, 3))[None]


# speedup vs baseline: 12.7813x; 1.0297x over previous
import functools

import jax
import jax.numpy as jnp
from jax import lax
from jax.experimental import pallas as pl
from jax.experimental.pallas import tpu as pltpu
from jax.experimental.pallas import tpu_sc as plsc

H, W, C = 64, 64, 256
PH, PW = 7, 7
N = 1000

NC, NS = 2, 16
NW = NC * NS
NSEG = PH * PW
CHUNK = 128
FULL_CPS = N // CHUNK
A_UNITS = NSEG * FULL_CPS
TAIL_START = FULL_CPS * CHUNK
TAIL = N - TAIL_START
A_PIPE = 10
TOTAL_ROWS = NSEG * N
NPAD = 1024


def _seg_indices(props_v, idx_v, par, s, p0, ngroups):
    i = (s * 9363) >> 16
    j = s - i * PW
    for g in range(ngroups):
        off = p0 + g * 16
        x0 = props_v[0, pl.ds(off, 16)]
        y0 = props_v[1, pl.ds(off, 16)]
        x1 = props_v[2, pl.ds(off, 16)]
        y1 = props_v[3, pl.ds(off, 16)]
        xmin = x0.astype(jnp.int32)
        ymin = y0.astype(jnp.int32)
        xt = x1.astype(jnp.int32)
        yt = y1.astype(jnp.int32)
        xmax = jnp.where(x1 > xt.astype(jnp.float32), xt + 1, xt)
        ymax = jnp.where(y1 > yt.astype(jnp.float32), yt + 1, yt)
        hh = jnp.maximum(ymax - ymin, 1)
        ww = jnp.maximum(xmax - xmin, 1)
        r = jnp.minimum(((2 * i + 1) * hh * 4682) >> 16, hh - 1) + ymin
        r = jnp.clip(r, 0, H - 1)
        c = jnp.minimum(((2 * j + 1) * ww * 4682) >> 16, ww - 1) + xmin
        c = jnp.clip(c, 0, W - 1)
        idx_v[par, pl.ds(g * 16, 16)] = r * W + c


def _a_unit(props_v, idx_v, u, par):
    q = (u * 1338) >> 16
    s = u - q * NSEG
    p0 = q * CHUNK
    _seg_indices(props_v, idx_v, par, s, p0, CHUNK // 16)
    return s * N + p0


def _b_unit(props_v, idx_v, s, par):
    _seg_indices(props_v, idx_v, par, s, TAIL_START, 7)
    return s * N + TAIL_START


def _body(fm, props, out, props_v, idx_v, gbuf, gsem):
    wid = lax.axis_index("s") * NC + lax.axis_index("c")
    pltpu.sync_copy(props, props_v)

    copies = [None] * (A_PIPE + 1)
    rows = [None] * (A_PIPE + 1)
    sizes = [CHUNK] * A_PIPE + [TAIL]
    for t in range(A_PIPE + 1):
        par = t % 2
        if t < A_PIPE:
            rows[t] = _a_unit(props_v, idx_v, wid + NW * t, par)
        else:
            rows[t] = _b_unit(props_v, idx_v, wid, par)
        copies[t] = pltpu.async_copy(
            fm.at[idx_v.at[par, pl.ds(0, sizes[t])]],
            gbuf.at[par, pl.ds(0, sizes[t])], gsem.at[par])
        if t > 0:
            copies[t - 1].wait()
            pltpu.sync_copy(gbuf.at[1 - par, pl.ds(0, sizes[t - 1])],
                            out.at[pl.ds(rows[t - 1], sizes[t - 1])])
    last = A_PIPE
    copies[last].wait()
    pltpu.sync_copy(gbuf.at[last % 2, pl.ds(0, sizes[last])],
                    out.at[pl.ds(rows[last], sizes[last])])

    ua = NW * A_PIPE + wid

    @pl.when(ua < A_UNITS)
    def _a_tail():
        row0 = _a_unit(props_v, idx_v, ua, 0)
        pltpu.async_copy(fm.at[idx_v.at[0]], gbuf.at[0], gsem.at[0]).wait()
        pltpu.sync_copy(gbuf.at[0], out.at[pl.ds(row0, CHUNK)])

    sb = NW + wid

    @pl.when(sb < NSEG)
    def _b_tail():
        row0 = _b_unit(props_v, idx_v, sb, 0)
        pltpu.async_copy(fm.at[idx_v.at[0, pl.ds(0, TAIL)]],
                         gbuf.at[0, pl.ds(0, TAIL)], gsem.at[0]).wait()
        pltpu.sync_copy(gbuf.at[0, pl.ds(0, TAIL)],
                        out.at[pl.ds(row0, TAIL)])


_sc_gather = functools.partial(
    pl.kernel,
    out_type=jax.ShapeDtypeStruct((TOTAL_ROWS, C), jnp.float32),
    mesh=plsc.VectorSubcoreMesh(
        core_axis_name="c", subcore_axis_name="s",
        num_cores=NC, num_subcores=NS),
    scratch_types=[
        pltpu.VMEM((4, NPAD), jnp.float32),
        pltpu.VMEM((2, CHUNK), jnp.int32),
        pltpu.VMEM((2, CHUNK, C), jnp.float32),
        pltpu.SemaphoreType.DMA((2,)),
    ],
    compiler_params=pltpu.CompilerParams(needs_layout_passes=False),
)(_body)


@jax.jit
def kernel(feature_map, proposals):
    fm = feature_map.reshape(H * W, C)
    props = jnp.zeros((4, NPAD), jnp.float32).at[:, :N].set(proposals[0].T)
    out = _sc_gather(fm, props)
    return jnp.transpose(
        out.reshape(PH, PW, N, C), (2, 0, 1, 3))[None]
